# baseline (speedup 1.0000x reference)
import jax
import jax.numpy as jnp
from jax import lax
from jax.experimental import pallas as pl
from jax.experimental.pallas import tpu as pltpu
from jax.experimental.pallas import tpu_sc as plsc

N = 10000
E = 320000
D = 128
H = 128
O = 64
B = 64

NC = 2
NS = 16
NT = NC * NS
EPT = 10240
KS = 2
REG = 5184
CAP = KS * REG
NEG = -3.0e38
ROWS = 64
RL = 256
AP = 72

_BLK = 2048


def _pre_body(xu_ref, wu_ref, asu_ref, adu_ref, bu_ref,
              xd_ref, wd_ref, asd_ref, add_ref, bd_ref,
              avu_ref, aduo_ref, selfu_ref, avd_ref, addo_ref, selfd_ref):
    hp = jax.lax.Precision.HIGHEST
    for (x_ref, w_ref, as_ref, ad_ref, b_ref, av_ref, ado_ref, self_ref) in (
        (xu_ref, wu_ref, asu_ref, adu_ref, bu_ref, avu_ref, aduo_ref, selfu_ref),
        (xd_ref, wd_ref, asd_ref, add_ref, bd_ref, avd_ref, addo_ref, selfd_ref),
    ):
        w = w_ref[...]
        a2 = jnp.concatenate([as_ref[...][:, None], ad_ref[...][:, None]],
                             axis=1)
        wa = jnp.dot(w, a2, preferred_element_type=jnp.float32, precision=hp)
        r = jnp.dot(x_ref[...], wa, preferred_element_type=jnp.float32,
                    precision=hp)
        av_ref[...] = r[:, 0]
        ado_ref[...] = r[:, 1]

        @pl.when(pl.program_id(0) == 0)
        def _():
            b = b_ref[...]
            iot = lax.broadcasted_iota(jnp.int32, (B, 1), 0)
            cnt = jnp.sum((b[None, :] <= iot).astype(jnp.int32), axis=1)
            self_ref[...] = cnt - 1


def _pre(xu, wu, asu, adu, bu, xd, wd, asd, add_, bd):
    grid = (pl.cdiv(N, _BLK),)
    xspec = pl.BlockSpec((_BLK, D), lambda i: (i, 0))
    wspec = pl.BlockSpec((D, H), lambda i: (0, 0))
    aspec = pl.BlockSpec((H,), lambda i: (0,))
    bspec = pl.BlockSpec((N,), lambda i: (0,))
    ospec = pl.BlockSpec((_BLK,), lambda i: (i,))
    sspec = pl.BlockSpec((B,), lambda i: (0,))
    return pl.pallas_call(
        _pre_body,
        grid=grid,
        in_specs=[xspec, wspec, aspec, aspec, bspec] * 2,
        out_specs=[ospec, ospec, sspec] * 2,
        out_shape=[
            jax.ShapeDtypeStruct((N,), jnp.float32),
            jax.ShapeDtypeStruct((N,), jnp.float32),
            jax.ShapeDtypeStruct((B,), jnp.int32),
        ] * 2,
    )(xu, wu, asu, adu, bu, xd, wd, asd, add_, bd)


def _sc_body(ue_hbm, de_hbm, avu_hbm, adu_hbm, avd_hbm,
             add_hbm, su_hbm, sd_hbm, xu_hbm, xd_hbm,
             accout, repout,
             asrc_v, adst_v, lut_v, self_v, adr_v, rep_v,
             sbuf, dbuf, ce_v, cs_v, ct_v, cl_v, accden_v, rows_v, sem):
    cid = lax.axis_index("c")
    sid = lax.axis_index("s")
    wid = sid * NC + cid
    lanes = lax.iota(jnp.int32, 16)

    def memset_body(i, _):
        for k in range(4):
            lut_v[pl.ds(i * 64 + k * 16, 16)] = jnp.full((16,), -1, jnp.int32)
        return 0

    lax.fori_loop(0, N // 64, memset_body, 0)
    lut_v[pl.ds(N - 16, 16)] = jnp.full((16,), -1, jnp.int32)

    def do_graph(g, e_hbm, av_hbm, ad_hbm, self_hbm, x_hbm):
        pltpu.sync_copy(av_hbm, asrc_v)
        pltpu.sync_copy(ad_hbm, adst_v)
        pltpu.sync_copy(self_hbm, self_v)

        for k in range(4):
            si = self_v[pl.ds(k * 16, 16)]
            si = jnp.where(si < 0, si + N, si)
            self_v[pl.ds(k * 16, 16)] = si
            plsc.store_scatter(lut_v, [si], lanes + k * 16)
        for k in range(4):
            si = self_v[pl.ds(k * 16, 16)]
            rep_v[pl.ds(k * 16, 16)] = plsc.load_gather(lut_v, [si])
            adr_v[pl.ds(k * 16, 16)] = plsc.load_gather(adst_v, [si])

        def zacc_body(i, _):
            for c in range(9):
                accden_v[i, pl.ds(c * 16, 16)] = jnp.zeros((16,), jnp.float32)
            return 0

        lax.fori_loop(0, B, zacc_body, 0)
        accden_v[B, pl.ds(0, 16)] = jnp.zeros((16,), jnp.float32)

        base_e = jnp.where(wid == NT - 1, E - EPT, wid * EPT)
        pltpu.sync_copy(e_hbm.at[0, pl.ds(base_e, EPT)], sbuf)
        pltpu.sync_copy(e_hbm.at[1, pl.ds(base_e, EPT)], dbuf)

        def s1(i, offs):
            new = []
            for k in range(KS):
                gi = i * KS + k
                dsts = dbuf[pl.ds(gi * 16, 16)]
                s = plsc.load_gather(lut_v, [dsts])
                msk = s >= 0
                plsc.store_compressed(ce_v.at[pl.ds(offs[k], 16)],
                                      lanes + gi * 16, mask=msk)
                cnt = plsc.all_reduce_population_count(msk)
                if getattr(cnt, "ndim", 0):
                    cnt = cnt[0]
                new.append(offs[k] + cnt)
            return tuple(new)

        i0 = jnp.where(wid == NT - 1,
                       jnp.int32((EPT - (E - (NT - 1) * EPT)) // (16 * KS)),
                       jnp.int32(0))
        offs = lax.fori_loop(
            i0, EPT // (16 * KS), s1,
            tuple(jnp.int32(k * REG) for k in range(KS)))
        counts = list(offs)
        tots = [k * REG + ((c - k * REG + 63) // 64) * 64
                for k, c in zip(range(KS), counts)]

        def s2(i, carry):
            mloc, count = carry
            gidx = lanes + i * 16
            val = gidx < count
            e16 = jnp.where(val, ce_v[pl.ds(i * 16, 16)], 0)
            srcs = plsc.load_gather(sbuf, [e16])
            dsts = plsc.load_gather(dbuf, [e16])
            s = plsc.load_gather(lut_v, [dsts])
            s0 = jnp.where(val, s, 0)
            a = plsc.load_gather(asrc_v, [srcs])
            adr = plsc.load_gather(adr_v, [s0])
            z = a + adr
            logit = jnp.where(z >= 0.0, z, jnp.float32(0.2) * z)
            logit = jnp.where(val, logit, NEG)
            cs_v[pl.ds(i * 16, 16)] = srcs
            ct_v[pl.ds(i * 16, 16)] = s0
            cl_v[pl.ds(i * 16, 16)] = logit
            return jnp.maximum(mloc, logit), count

        mloc = jnp.full((16,), NEG, jnp.float32)
        for k in range(KS):
            mloc, _ = lax.fori_loop(k * REG // 16, tots[k] // 16, s2,
                                    (mloc, counts[k]))

        mmax = jnp.max(mloc)
        accden_v[B, pl.ds(0, 16)] = jnp.full((16,), mmax, jnp.float32)

        def exb(i, count):
            l = cl_v[pl.ds(i * 16, 16)]
            e = jnp.exp(l - mmax)
            gidx = lanes + i * 16
            e = jnp.where(gidx < count, e, jnp.float32(0.0))
            cl_v[pl.ds(i * 16, 16)] = e
            return count

        lane0 = lanes == 0

        def ch3(c, count):
            pltpu.async_copy(
                x_hbm.at[cs_v.at[pl.ds(c * ROWS, ROWS)]], rows_v, sem
            ).wait()
            rem = jnp.minimum(ROWS, count - c * ROWS)

            def edge(i, _):
                eidx = jnp.full((16,), c * ROWS + i, jnp.int32)
                exv = plsc.load_gather(cl_v, [eidx])
                slv = plsc.load_gather(ct_v, [eidx])
                sl = slv[0]
                plsc.addupdate(accden_v.at[sl, pl.ds(128, 16)],
                               jnp.where(lane0, exv, jnp.float32(0.0)))
                for h8 in range(8):
                    plsc.addupdate(accden_v.at[sl, pl.ds(h8 * 16, 16)],
                                   exv * rows_v[i, pl.ds(h8 * 16, 16)])
                return 0

            lax.fori_loop(0, rem, edge, 0)
            return count

        for k in range(KS):
            lax.fori_loop(k * REG // 16, tots[k] // 16, exb, counts[k])
            lax.fori_loop(k * REG // ROWS, tots[k] // ROWS, ch3, counts[k])

        pltpu.sync_copy(accden_v, accout.at[g, pl.ds(wid * AP, AP)])

        @pl.when(jnp.logical_and(sid == 0, cid == 0))
        def _():
            pltpu.sync_copy(rep_v, repout.at[g])

        for k in range(4):
            si = self_v[pl.ds(k * 16, 16)]
            plsc.store_scatter(lut_v, [si], jnp.full((16,), -1, jnp.int32))

    do_graph(0, ue_hbm, avu_hbm, adu_hbm, su_hbm, xu_hbm)
    do_graph(1, de_hbm, avd_hbm, add_hbm, sd_hbm, xd_hbm)


def _sc_main(ue, de, avu, adu, avd, add_, su, sd, xu, xd):
    mesh = plsc.VectorSubcoreMesh(core_axis_name="c", subcore_axis_name="s")
    f = pl.kernel(
        _sc_body,
        out_type=[
            jax.ShapeDtypeStruct((2, NT * AP, RL), jnp.float32),
            jax.ShapeDtypeStruct((2, B), jnp.int32),
        ],
        mesh=mesh,
        compiler_params=pltpu.CompilerParams(needs_layout_passes=False),
        scratch_types=[
            pltpu.VMEM((N,), jnp.float32),
            pltpu.VMEM((N,), jnp.float32),
            pltpu.VMEM((N,), jnp.int32),
            pltpu.VMEM((B,), jnp.int32),
            pltpu.VMEM((B,), jnp.float32),
            pltpu.VMEM((B,), jnp.int32),
            pltpu.VMEM((EPT,), jnp.int32),
            pltpu.VMEM((EPT,), jnp.int32),
            pltpu.VMEM((CAP,), jnp.int32),
            pltpu.VMEM((CAP,), jnp.int32),
            pltpu.VMEM((CAP,), jnp.int32),
            pltpu.VMEM((CAP,), jnp.float32),
            pltpu.VMEM((AP, RL), jnp.float32),
            pltpu.VMEM((ROWS, D), jnp.float32),
            pltpu.SemaphoreType.DMA,
        ],
    )
    return f(ue, de, avu, adu, avd, add_, su, sd, xu, xd)


def _post_body(acc_ref, rep_ref, wu_ref, bu_ref, wd_ref, bd_ref, mw_ref,
               mb_ref, o_ref):
    x = acc_ref[...].reshape(2, NT, AP, RL)
    r = rep_ref[...]
    iot = lax.broadcasted_iota(jnp.int32, (B, B), 1)
    embs = []
    for g, (w_ref, b_ref) in enumerate(((wu_ref, bu_ref), (wd_ref, bd_ref))):
        m_w = x[g, :, B:B + 1, 0:1]
        mm = jnp.max(m_w, axis=0, keepdims=True)
        sc = jnp.exp(m_w - mm)
        acc = jnp.sum(x[g, :, 0:B, 0:128] * sc, axis=0)
        den = jnp.sum(x[g, :, 0:B, 128:129] * sc, axis=0)
        rows = acc / (den + jnp.float32(1e-16))
        out = jnp.dot(rows, w_ref[...], preferred_element_type=jnp.float32,
                      precision=jax.lax.Precision.HIGHEST)
        out = out + b_ref[...][None, :]
        sig = jax.nn.sigmoid(out)
        p = (r[g][:, None] == iot).astype(jnp.float32)
        embs.append(jnp.dot(p, sig, preferred_element_type=jnp.float32))
    prod = embs[0] * embs[1]
    o_ref[...] = (
        jnp.dot(prod, mw_ref[...], preferred_element_type=jnp.float32)
        + mb_ref[...][None, :]
    )


def _post(accden, rep, w_up, b_up, w_down, b_down, mlp_w, mlp_b):
    return pl.pallas_call(
        _post_body,
        out_shape=jax.ShapeDtypeStruct((B, O), jnp.float32),
    )(accden, rep, w_up, b_up, w_down, b_down, mlp_w, mlp_b)


def kernel(up_x, up_edge_index, up_batch, down_x, down_edge_index, down_batch,
           W_up, att_src_up, att_dst_up, bias_up,
           W_down, att_src_down, att_dst_down, bias_down, mlp_W, mlp_b):
    avu, adu, su, avd, add_, sd = _pre(
        up_x, W_up, att_src_up, att_dst_up, up_batch,
        down_x, W_down, att_src_down, att_dst_down, down_batch)
    accden, rep = _sc_main(
        up_edge_index, down_edge_index,
        avu, adu, avd, add_, su, sd, up_x, down_x,
    )
    return _post(accden, rep, W_up, bias_up, W_down, bias_down, mlp_W, mlp_b)


# speedup vs baseline: 1.1612x; 1.1612x over previous
import jax
import jax.numpy as jnp
from jax import lax
from jax.experimental import pallas as pl
from jax.experimental.pallas import tpu as pltpu
from jax.experimental.pallas import tpu_sc as plsc

N = 10000
E = 320000
D = 128
H = 128
O = 64
B = 64

NC = 2
NS = 16
NT = NC * NS
EPT = 10240
KS = 2
REG = 5184
CAP = KS * REG
NEG = -3.0e38
ROWS = 64
RL = 256
AP = 72

_BLK = 2048


def _pre_body(xu_ref, wu_ref, asu_ref, adu_ref, bu_ref,
              xd_ref, wd_ref, asd_ref, add_ref, bd_ref,
              avu_ref, aduo_ref, selfu_ref, avd_ref, addo_ref, selfd_ref):
    hp = jax.lax.Precision.HIGHEST
    for (x_ref, w_ref, as_ref, ad_ref, b_ref, av_ref, ado_ref, self_ref) in (
        (xu_ref, wu_ref, asu_ref, adu_ref, bu_ref, avu_ref, aduo_ref, selfu_ref),
        (xd_ref, wd_ref, asd_ref, add_ref, bd_ref, avd_ref, addo_ref, selfd_ref),
    ):
        w = w_ref[...]
        a2 = jnp.concatenate([as_ref[...][:, None], ad_ref[...][:, None]],
                             axis=1)
        wa = jnp.dot(w, a2, preferred_element_type=jnp.float32, precision=hp)
        r = jnp.dot(x_ref[...], wa, preferred_element_type=jnp.float32)
        av_ref[...] = r[:, 0]
        ado_ref[...] = r[:, 1]

        @pl.when(pl.program_id(0) == 0)
        def _():
            b = b_ref[...]
            iot = lax.broadcasted_iota(jnp.int32, (B, 1), 0)
            cnt = jnp.sum((b[None, :] <= iot).astype(jnp.int32), axis=1)
            self_ref[...] = cnt - 1


def _pre(xu, wu, asu, adu, bu, xd, wd, asd, add_, bd):
    grid = (pl.cdiv(N, _BLK),)
    xspec = pl.BlockSpec((_BLK, D), lambda i: (i, 0))
    wspec = pl.BlockSpec((D, H), lambda i: (0, 0))
    aspec = pl.BlockSpec((H,), lambda i: (0,))
    bspec = pl.BlockSpec((N,), lambda i: (0,))
    ospec = pl.BlockSpec((_BLK,), lambda i: (i,))
    sspec = pl.BlockSpec((B,), lambda i: (0,))
    return pl.pallas_call(
        _pre_body,
        grid=grid,
        in_specs=[xspec, wspec, aspec, aspec, bspec] * 2,
        out_specs=[ospec, ospec, sspec] * 2,
        out_shape=[
            jax.ShapeDtypeStruct((N,), jnp.float32),
            jax.ShapeDtypeStruct((N,), jnp.float32),
            jax.ShapeDtypeStruct((B,), jnp.int32),
        ] * 2,
    )(xu, wu, asu, adu, bu, xd, wd, asd, add_, bd)


def _sc_body(ue_hbm, de_hbm, avu_hbm, adu_hbm, avd_hbm,
             add_hbm, su_hbm, sd_hbm, xu_hbm, xd_hbm,
             accout, repout,
             asrc_v, adst_v, lut_v, self_v, adr_v, rep_v,
             sbuf, dbuf, ce_v, cs_v, ct_v, cl_v, accden_v, rows_v, sem):
    cid = lax.axis_index("c")
    sid = lax.axis_index("s")
    wid = sid * NC + cid
    lanes = lax.iota(jnp.int32, 16)

    def memset_body(i, _):
        for k in range(4):
            lut_v[pl.ds(i * 64 + k * 16, 16)] = jnp.full((16,), -1, jnp.int32)
        return 0

    lax.fori_loop(0, N // 64, memset_body, 0)
    lut_v[pl.ds(N - 16, 16)] = jnp.full((16,), -1, jnp.int32)

    def do_graph(g, e_hbm, av_hbm, ad_hbm, self_hbm, x_hbm):
        pltpu.sync_copy(av_hbm, asrc_v)
        pltpu.sync_copy(ad_hbm, adst_v)
        pltpu.sync_copy(self_hbm, self_v)

        for k in range(4):
            si = self_v[pl.ds(k * 16, 16)]
            si = jnp.where(si < 0, si + N, si)
            self_v[pl.ds(k * 16, 16)] = si
            plsc.store_scatter(lut_v, [si], lanes + k * 16)
        for k in range(4):
            si = self_v[pl.ds(k * 16, 16)]
            rep_v[pl.ds(k * 16, 16)] = plsc.load_gather(lut_v, [si])
            adr_v[pl.ds(k * 16, 16)] = plsc.load_gather(adst_v, [si])

        def zacc_body(i, _):
            for c in range(9):
                accden_v[i, pl.ds(c * 16, 16)] = jnp.zeros((16,), jnp.float32)
            return 0

        lax.fori_loop(0, B, zacc_body, 0)
        accden_v[B, pl.ds(0, 16)] = jnp.zeros((16,), jnp.float32)

        base_e = jnp.where(wid == NT - 1, E - EPT, wid * EPT)
        pltpu.sync_copy(e_hbm.at[0, pl.ds(base_e, EPT)], sbuf)
        pltpu.sync_copy(e_hbm.at[1, pl.ds(base_e, EPT)], dbuf)

        U = 2

        def s1(i, offs):
            new = list(offs)
            for u in range(U):
                for k in range(KS):
                    gi = (i * U + u) * KS + k
                    dsts = dbuf[pl.ds(gi * 16, 16)]
                    s = plsc.load_gather(lut_v, [dsts])
                    msk = s >= 0
                    plsc.store_compressed(ce_v.at[pl.ds(new[k], 16)],
                                          lanes + gi * 16, mask=msk)
                    cnt = plsc.all_reduce_population_count(msk)
                    if getattr(cnt, "ndim", 0):
                        cnt = cnt[0]
                    new[k] = new[k] + cnt
            return tuple(new)

        i0 = jnp.where(wid == NT - 1,
                       jnp.int32((EPT - (E - (NT - 1) * EPT)) // (16 * KS * U)),
                       jnp.int32(0))
        offs = lax.fori_loop(
            i0, EPT // (16 * KS * U), s1,
            tuple(jnp.int32(k * REG) for k in range(KS)))
        counts = list(offs)
        tots = [k * REG + ((c - k * REG + 63) // 64) * 64
                for k, c in zip(range(KS), counts)]

        def s2(i, carry):
            mloc, count = carry
            gidx = lanes + i * 16
            val = gidx < count
            e16 = jnp.where(val, ce_v[pl.ds(i * 16, 16)], 0)
            srcs = plsc.load_gather(sbuf, [e16])
            dsts = plsc.load_gather(dbuf, [e16])
            s = plsc.load_gather(lut_v, [dsts])
            s0 = jnp.where(val, s, 0)
            a = plsc.load_gather(asrc_v, [srcs])
            adr = plsc.load_gather(adr_v, [s0])
            z = a + adr
            logit = jnp.where(z >= 0.0, z, jnp.float32(0.2) * z)
            logit = jnp.where(val, logit, NEG)
            cs_v[pl.ds(i * 16, 16)] = srcs
            ct_v[pl.ds(i * 16, 16)] = s0
            cl_v[pl.ds(i * 16, 16)] = logit
            return jnp.maximum(mloc, logit), count

        mloc = jnp.full((16,), NEG, jnp.float32)
        for k in range(KS):
            mloc, _ = lax.fori_loop(k * REG // 16, tots[k] // 16, s2,
                                    (mloc, counts[k]))

        mmax = jnp.max(mloc)
        accden_v[B, pl.ds(0, 16)] = jnp.full((16,), mmax, jnp.float32)

        def exb(i, count):
            l = cl_v[pl.ds(i * 16, 16)]
            e = jnp.exp(l - mmax)
            gidx = lanes + i * 16
            e = jnp.where(gidx < count, e, jnp.float32(0.0))
            cl_v[pl.ds(i * 16, 16)] = e
            return count

        lane0 = lanes == 0

        def ch3(c, count):
            pltpu.async_copy(
                x_hbm.at[cs_v.at[pl.ds(c * ROWS, ROWS)]], rows_v, sem
            ).wait()
            rem = jnp.minimum(ROWS, count - c * ROWS)

            def edge(i, _):
                eidx = jnp.full((16,), c * ROWS + i, jnp.int32)
                exv = plsc.load_gather(cl_v, [eidx])
                slv = plsc.load_gather(ct_v, [eidx])
                sl = slv[0]
                plsc.addupdate(accden_v.at[sl, pl.ds(128, 16)],
                               jnp.where(lane0, exv, jnp.float32(0.0)))
                for h8 in range(8):
                    plsc.addupdate(accden_v.at[sl, pl.ds(h8 * 16, 16)],
                                   exv * rows_v[i, pl.ds(h8 * 16, 16)])
                return 0

            lax.fori_loop(0, rem, edge, 0)
            return count

        for k in range(KS):
            lax.fori_loop(k * REG // 16, tots[k] // 16, exb, counts[k])
            lax.fori_loop(k * REG // ROWS, tots[k] // ROWS, ch3, counts[k])

        pltpu.sync_copy(accden_v, accout.at[g, pl.ds(wid * AP, AP)])

        @pl.when(jnp.logical_and(sid == 0, cid == 0))
        def _():
            pltpu.sync_copy(rep_v, repout.at[g])

        for k in range(4):
            si = self_v[pl.ds(k * 16, 16)]
            plsc.store_scatter(lut_v, [si], jnp.full((16,), -1, jnp.int32))

    do_graph(0, ue_hbm, avu_hbm, adu_hbm, su_hbm, xu_hbm)
    do_graph(1, de_hbm, avd_hbm, add_hbm, sd_hbm, xd_hbm)


def _sc_main(ue, de, avu, adu, avd, add_, su, sd, xu, xd):
    mesh = plsc.VectorSubcoreMesh(core_axis_name="c", subcore_axis_name="s")
    f = pl.kernel(
        _sc_body,
        out_type=[
            jax.ShapeDtypeStruct((2, NT * AP, RL), jnp.float32),
            jax.ShapeDtypeStruct((2, B), jnp.int32),
        ],
        mesh=mesh,
        compiler_params=pltpu.CompilerParams(needs_layout_passes=False),
        scratch_types=[
            pltpu.VMEM((N,), jnp.float32),
            pltpu.VMEM((N,), jnp.float32),
            pltpu.VMEM((N,), jnp.int32),
            pltpu.VMEM((B,), jnp.int32),
            pltpu.VMEM((B,), jnp.float32),
            pltpu.VMEM((B,), jnp.int32),
            pltpu.VMEM((EPT,), jnp.int32),
            pltpu.VMEM((EPT,), jnp.int32),
            pltpu.VMEM((CAP,), jnp.int32),
            pltpu.VMEM((CAP,), jnp.int32),
            pltpu.VMEM((CAP,), jnp.int32),
            pltpu.VMEM((CAP,), jnp.float32),
            pltpu.VMEM((AP, RL), jnp.float32),
            pltpu.VMEM((ROWS, D), jnp.float32),
            pltpu.SemaphoreType.DMA,
        ],
    )
    return f(ue, de, avu, adu, avd, add_, su, sd, xu, xd)


def _post_body(acc_ref, rep_ref, wu_ref, bu_ref, wd_ref, bd_ref, mw_ref,
               mb_ref, o_ref):
    x = acc_ref[...].reshape(2, NT, AP, RL)
    r = rep_ref[...]
    iot = lax.broadcasted_iota(jnp.int32, (B, B), 1)
    embs = []
    for g, (w_ref, b_ref) in enumerate(((wu_ref, bu_ref), (wd_ref, bd_ref))):
        m_w = x[g, :, B:B + 1, 0:1]
        mm = jnp.max(m_w, axis=0, keepdims=True)
        sc = jnp.exp(m_w - mm)
        acc = jnp.sum(x[g, :, 0:B, 0:128] * sc, axis=0)
        den = jnp.sum(x[g, :, 0:B, 128:129] * sc, axis=0)
        rows = acc / (den + jnp.float32(1e-16))
        out = jnp.dot(rows, w_ref[...], preferred_element_type=jnp.float32,
                      precision=jax.lax.Precision.HIGHEST)
        out = out + b_ref[...][None, :]
        sig = jax.nn.sigmoid(out)
        p = (r[g][:, None] == iot).astype(jnp.float32)
        embs.append(jnp.dot(p, sig, preferred_element_type=jnp.float32))
    prod = embs[0] * embs[1]
    o_ref[...] = (
        jnp.dot(prod, mw_ref[...], preferred_element_type=jnp.float32)
        + mb_ref[...][None, :]
    )


def _post(accden, rep, w_up, b_up, w_down, b_down, mlp_w, mlp_b):
    return pl.pallas_call(
        _post_body,
        out_shape=jax.ShapeDtypeStruct((B, O), jnp.float32),
    )(accden, rep, w_up, b_up, w_down, b_down, mlp_w, mlp_b)


def kernel(up_x, up_edge_index, up_batch, down_x, down_edge_index, down_batch,
           W_up, att_src_up, att_dst_up, bias_up,
           W_down, att_src_down, att_dst_down, bias_down, mlp_W, mlp_b):
    avu, adu, su, avd, add_, sd = _pre(
        up_x, W_up, att_src_up, att_dst_up, up_batch,
        down_x, W_down, att_src_down, att_dst_down, down_batch)
    accden, rep = _sc_main(
        up_edge_index, down_edge_index,
        avu, adu, avd, add_, su, sd, up_x, down_x,
    )
    return _post(accden, rep, W_up, bias_up, W_down, bias_down, mlp_W, mlp_b)


# speedup vs baseline: 1.3398x; 1.1538x over previous
import jax
import jax.numpy as jnp
from jax import lax
from jax.experimental import pallas as pl
from jax.experimental.pallas import tpu as pltpu
from jax.experimental.pallas import tpu_sc as plsc

N = 10000
E = 320000
D = 128
H = 128
O = 64
B = 64

NC = 2
NS = 16
NT = NC * NS
EPT = 10240
KS = 2
REG = 5184
CAP = KS * REG
NEG = -3.0e38
ROWS = 64
RL = 256
AP = 72

_BLK = 2048


def _pre_body(xu_ref, wu_ref, asu_ref, adu_ref, bu_ref,
              xd_ref, wd_ref, asd_ref, add_ref, bd_ref,
              avu_ref, aduo_ref, selfu_ref, avd_ref, addo_ref, selfd_ref):
    hp = jax.lax.Precision.HIGHEST
    for (x_ref, w_ref, as_ref, ad_ref, b_ref, av_ref, ado_ref, self_ref) in (
        (xu_ref, wu_ref, asu_ref, adu_ref, bu_ref, avu_ref, aduo_ref, selfu_ref),
        (xd_ref, wd_ref, asd_ref, add_ref, bd_ref, avd_ref, addo_ref, selfd_ref),
    ):
        w = w_ref[...]
        a2 = jnp.concatenate([as_ref[...][:, None], ad_ref[...][:, None]],
                             axis=1)
        wa = jnp.dot(w, a2, preferred_element_type=jnp.float32, precision=hp)
        r = jnp.dot(x_ref[...], wa, preferred_element_type=jnp.float32)
        av_ref[...] = r[:, 0]
        ado_ref[...] = r[:, 1]

        @pl.when(pl.program_id(0) == 0)
        def _():
            b = b_ref[...]
            iot = lax.broadcasted_iota(jnp.int32, (B, 1), 0)
            cnt = jnp.sum((b[None, :] <= iot).astype(jnp.int32), axis=1)
            self_ref[...] = cnt - 1


def _pre(xu, wu, asu, adu, bu, xd, wd, asd, add_, bd):
    grid = (pl.cdiv(N, _BLK),)
    xspec = pl.BlockSpec((_BLK, D), lambda i: (i, 0))
    wspec = pl.BlockSpec((D, H), lambda i: (0, 0))
    aspec = pl.BlockSpec((H,), lambda i: (0,))
    bspec = pl.BlockSpec((N,), lambda i: (0,))
    ospec = pl.BlockSpec((_BLK,), lambda i: (i,))
    sspec = pl.BlockSpec((B,), lambda i: (0,))
    return pl.pallas_call(
        _pre_body,
        grid=grid,
        in_specs=[xspec, wspec, aspec, aspec, bspec] * 2,
        out_specs=[ospec, ospec, sspec] * 2,
        out_shape=[
            jax.ShapeDtypeStruct((N,), jnp.float32),
            jax.ShapeDtypeStruct((N,), jnp.float32),
            jax.ShapeDtypeStruct((B,), jnp.int32),
        ] * 2,
    )(xu, wu, asu, adu, bu, xd, wd, asd, add_, bd)


def _sc_body(ue_hbm, de_hbm, avu_hbm, adu_hbm, avd_hbm,
             add_hbm, su_hbm, sd_hbm, xu_hbm, xd_hbm,
             accout, repout,
             asrc_v, adst_v, lut_v, self_v, adr_v, rep_v,
             sbuf, dbuf, ce_v, cs_v, ct_v, cl_v, accden_v, rows_v, sem):
    cid = lax.axis_index("c")
    sid = lax.axis_index("s")
    wid = sid * NC + cid
    lanes = lax.iota(jnp.int32, 16)

    def memset_body(i, _):
        for k in range(4):
            lut_v[pl.ds(i * 64 + k * 16, 16)] = jnp.full((16,), -1, jnp.int32)
        return 0

    lax.fori_loop(0, N // 64, memset_body, 0)
    lut_v[pl.ds(N - 16, 16)] = jnp.full((16,), -1, jnp.int32)

    def do_graph(g, e_hbm, av_hbm, ad_hbm, self_hbm, x_hbm):
        pltpu.sync_copy(av_hbm, asrc_v)
        pltpu.sync_copy(ad_hbm, adst_v)
        pltpu.sync_copy(self_hbm, self_v)

        for k in range(4):
            si = self_v[pl.ds(k * 16, 16)]
            si = jnp.where(si < 0, si + N, si)
            self_v[pl.ds(k * 16, 16)] = si
            plsc.store_scatter(lut_v, [si], lanes + k * 16)
        for k in range(4):
            si = self_v[pl.ds(k * 16, 16)]
            rep_v[pl.ds(k * 16, 16)] = plsc.load_gather(lut_v, [si])
            adr_v[pl.ds(k * 16, 16)] = plsc.load_gather(adst_v, [si])

        def zacc_body(i, _):
            for c in range(9):
                accden_v[i, pl.ds(c * 16, 16)] = jnp.zeros((16,), jnp.float32)
            return 0

        lax.fori_loop(0, B, zacc_body, 0)
        accden_v[B, pl.ds(0, 16)] = jnp.zeros((16,), jnp.float32)

        base_e = jnp.where(wid == NT - 1, E - EPT, wid * EPT)
        pltpu.sync_copy(e_hbm.at[0, pl.ds(base_e, EPT)], sbuf)
        pltpu.sync_copy(e_hbm.at[1, pl.ds(base_e, EPT)], dbuf)

        def s1(i, offs):
            new = list(offs)
            for k in range(KS):
                dsts = dbuf[pl.ds(i + k * 16, 16)]
                s = plsc.load_gather(lut_v, [dsts])
                msk = s >= 0
                plsc.store_compressed(ce_v.at[pl.ds(new[k], 16)],
                                      lanes + i + k * 16, mask=msk)
                cnt = plsc.all_reduce_population_count(msk)
                if getattr(cnt, "ndim", 0):
                    cnt = cnt[0]
                new[k] = new[k] + cnt
            return tuple(new)

        i0 = jnp.where(wid == NT - 1,
                       jnp.int32(EPT - (E - (NT - 1) * EPT)), jnp.int32(0))
        offs = plsc.parallel_loop(
            i0, EPT, 16 * KS, unroll=4,
            carry=tuple(jnp.int32(k * REG) for k in range(KS)))(s1)
        counts = list(offs)
        tots = [k * REG + ((c - k * REG + 63) // 64) * 64
                for k, c in zip(range(KS), counts)]

        def s2(i, carry):
            mloc, count = carry
            gidx = lanes + i * 16
            val = gidx < count
            e16 = jnp.where(val, ce_v[pl.ds(i * 16, 16)], 0)
            srcs = plsc.load_gather(sbuf, [e16])
            dsts = plsc.load_gather(dbuf, [e16])
            s = plsc.load_gather(lut_v, [dsts])
            s0 = jnp.where(val, s, 0)
            a = plsc.load_gather(asrc_v, [srcs])
            adr = plsc.load_gather(adr_v, [s0])
            z = a + adr
            logit = jnp.where(z >= 0.0, z, jnp.float32(0.2) * z)
            logit = jnp.where(val, logit, NEG)
            cs_v[pl.ds(i * 16, 16)] = srcs
            ct_v[pl.ds(i * 16, 16)] = s0
            cl_v[pl.ds(i * 16, 16)] = logit
            return jnp.maximum(mloc, logit), count

        mloc = jnp.full((16,), NEG, jnp.float32)
        for k in range(KS):
            mloc, _ = lax.fori_loop(k * REG // 16, tots[k] // 16, s2,
                                    (mloc, counts[k]))

        mmax = jnp.max(mloc)
        accden_v[B, pl.ds(0, 16)] = jnp.full((16,), mmax, jnp.float32)

        def exb(i, count):
            l = cl_v[pl.ds(i * 16, 16)]
            e = jnp.exp(l - mmax)
            gidx = lanes + i * 16
            e = jnp.where(gidx < count, e, jnp.float32(0.0))
            cl_v[pl.ds(i * 16, 16)] = e
            return count

        lane0 = lanes == 0

        def ch3(c, count):
            pltpu.async_copy(
                x_hbm.at[cs_v.at[pl.ds(c * ROWS, ROWS)]], rows_v, sem
            ).wait()
            rem = jnp.minimum(ROWS, count - c * ROWS)

            def edge(i, _):
                eidx = jnp.full((16,), c * ROWS + i, jnp.int32)
                exv = plsc.load_gather(cl_v, [eidx])
                slv = plsc.load_gather(ct_v, [eidx])
                sl = slv[0]
                plsc.addupdate(accden_v.at[sl, pl.ds(128, 16)],
                               jnp.where(lane0, exv, jnp.float32(0.0)))
                for h8 in range(8):
                    plsc.addupdate(accden_v.at[sl, pl.ds(h8 * 16, 16)],
                                   exv * rows_v[i, pl.ds(h8 * 16, 16)])
                return 0

            lax.fori_loop(0, rem, edge, 0)
            return count

        for k in range(KS):
            lax.fori_loop(k * REG // 16, tots[k] // 16, exb, counts[k])
            lax.fori_loop(k * REG // ROWS, tots[k] // ROWS, ch3, counts[k])

        pltpu.sync_copy(accden_v, accout.at[g, pl.ds(wid * AP, AP)])

        @pl.when(jnp.logical_and(sid == 0, cid == 0))
        def _():
            pltpu.sync_copy(rep_v, repout.at[g])

        for k in range(4):
            si = self_v[pl.ds(k * 16, 16)]
            plsc.store_scatter(lut_v, [si], jnp.full((16,), -1, jnp.int32))

    do_graph(0, ue_hbm, avu_hbm, adu_hbm, su_hbm, xu_hbm)
    do_graph(1, de_hbm, avd_hbm, add_hbm, sd_hbm, xd_hbm)


def _sc_main(ue, de, avu, adu, avd, add_, su, sd, xu, xd):
    mesh = plsc.VectorSubcoreMesh(core_axis_name="c", subcore_axis_name="s")
    f = pl.kernel(
        _sc_body,
        out_type=[
            jax.ShapeDtypeStruct((2, NT * AP, RL), jnp.float32),
            jax.ShapeDtypeStruct((2, B), jnp.int32),
        ],
        mesh=mesh,
        compiler_params=pltpu.CompilerParams(needs_layout_passes=False),
        scratch_types=[
            pltpu.VMEM((N,), jnp.float32),
            pltpu.VMEM((N,), jnp.float32),
            pltpu.VMEM((N,), jnp.int32),
            pltpu.VMEM((B,), jnp.int32),
            pltpu.VMEM((B,), jnp.float32),
            pltpu.VMEM((B,), jnp.int32),
            pltpu.VMEM((EPT,), jnp.int32),
            pltpu.VMEM((EPT,), jnp.int32),
            pltpu.VMEM((CAP,), jnp.int32),
            pltpu.VMEM((CAP,), jnp.int32),
            pltpu.VMEM((CAP,), jnp.int32),
            pltpu.VMEM((CAP,), jnp.float32),
            pltpu.VMEM((AP, RL), jnp.float32),
            pltpu.VMEM((ROWS, D), jnp.float32),
            pltpu.SemaphoreType.DMA,
        ],
    )
    return f(ue, de, avu, adu, avd, add_, su, sd, xu, xd)


def _post_body(acc_ref, rep_ref, wu_ref, bu_ref, wd_ref, bd_ref, mw_ref,
               mb_ref, o_ref):
    x = acc_ref[...].reshape(2, NT, AP, RL)
    r = rep_ref[...]
    iot = lax.broadcasted_iota(jnp.int32, (B, B), 1)
    embs = []
    for g, (w_ref, b_ref) in enumerate(((wu_ref, bu_ref), (wd_ref, bd_ref))):
        m_w = x[g, :, B:B + 1, 0:1]
        mm = jnp.max(m_w, axis=0, keepdims=True)
        sc = jnp.exp(m_w - mm)
        acc = jnp.sum(x[g, :, 0:B, 0:128] * sc, axis=0)
        den = jnp.sum(x[g, :, 0:B, 128:129] * sc, axis=0)
        rows = acc / (den + jnp.float32(1e-16))
        out = jnp.dot(rows, w_ref[...], preferred_element_type=jnp.float32,
                      precision=jax.lax.Precision.HIGHEST)
        out = out + b_ref[...][None, :]
        sig = jax.nn.sigmoid(out)
        p = (r[g][:, None] == iot).astype(jnp.float32)
        embs.append(jnp.dot(p, sig, preferred_element_type=jnp.float32))
    prod = embs[0] * embs[1]
    o_ref[...] = (
        jnp.dot(prod, mw_ref[...], preferred_element_type=jnp.float32)
        + mb_ref[...][None, :]
    )


def _post(accden, rep, w_up, b_up, w_down, b_down, mlp_w, mlp_b):
    return pl.pallas_call(
        _post_body,
        out_shape=jax.ShapeDtypeStruct((B, O), jnp.float32),
    )(accden, rep, w_up, b_up, w_down, b_down, mlp_w, mlp_b)


def kernel(up_x, up_edge_index, up_batch, down_x, down_edge_index, down_batch,
           W_up, att_src_up, att_dst_up, bias_up,
           W_down, att_src_down, att_dst_down, bias_down, mlp_W, mlp_b):
    avu, adu, su, avd, add_, sd = _pre(
        up_x, W_up, att_src_up, att_dst_up, up_batch,
        down_x, W_down, att_src_down, att_dst_down, down_batch)
    accden, rep = _sc_main(
        up_edge_index, down_edge_index,
        avu, adu, avd, add_, su, sd, up_x, down_x,
    )
    return _post(accden, rep, W_up, bias_up, W_down, bias_down, mlp_W, mlp_b)
